# baseline (speedup 1.0000x reference)
import jax
import jax.numpy as jnp
from jax.experimental import pallas as pl


_N_BLK = 2048
_N_VPU = 512


def _nnd_body(x_ref, yt_ref, d1_ref, d2_ref):
    nb = pl.program_id(1)
    x = x_ref[0]
    yt = yt_ref[0]

    n_blk = x.shape[0]
    m = yt.shape[1]
    bf16, f32 = jnp.bfloat16, jnp.float32

    xm = ((-2.0) * x).astype(bf16)
    yb = yt.astype(bf16)
    x2 = jnp.sum(x * x, axis=1, keepdims=True)
    y2 = jnp.sum(yt * yt, axis=0, keepdims=True)
    x2h = x2.astype(bf16)
    x2l = (x2 - x2h.astype(f32)).astype(bf16)
    y2h = y2.astype(bf16)
    y2l = (y2 - y2h.astype(f32)).astype(bf16)

    n_mxu = n_blk - _N_VPU
    lhs = jnp.concatenate(
        [xm[:n_mxu], x2h[:n_mxu], x2l[:n_mxu],
         jnp.ones((n_mxu, 1), bf16), jnp.ones((n_mxu, 1), bf16)], axis=1)
    rhs = jnp.concatenate(
        [yb, jnp.ones((1, m), bf16), jnp.ones((1, m), bf16),
         y2h, y2l], axis=0)

    da = jax.lax.dot_general(lhs, rhs, (((1,), (0,)), ((), ())),
                             preferred_element_type=f32)

    xv = xm[n_mxu:].astype(f32)
    yv = yb.astype(f32)
    acc = y2
    for k in range(3):
        acc = acc + xv[:, k:k + 1] * yv[k:k + 1, :]
    db = acc + x2[n_mxu:]

    d1_ref[0, :n_mxu] = jnp.maximum(jnp.min(da, axis=1, keepdims=True), 0.0)
    d1_ref[0, n_mxu:] = jnp.maximum(jnp.min(db, axis=1, keepdims=True), 0.0)

    cur = jnp.maximum(
        jnp.minimum(jnp.min(da, axis=0, keepdims=True),
                    jnp.min(db, axis=0, keepdims=True)), 0.0)

    @pl.when(nb == 0)
    def _init():
        d2_ref[0] = cur

    @pl.when(nb != 0)
    def _accum():
        d2_ref[0] = jnp.minimum(d2_ref[0], cur)


def kernel(input1, input2):
    B, N, _ = input1.shape
    M = input2.shape[1]
    yt = jnp.transpose(input2, (0, 2, 1))

    nb = N // _N_BLK
    out1, out2 = pl.pallas_call(
        _nnd_body,
        grid=(B, nb),
        in_specs=[
            pl.BlockSpec((1, _N_BLK, 3), lambda b, i: (b, i, 0)),
            pl.BlockSpec((1, 3, M), lambda b, i: (b, 0, 0)),
        ],
        out_specs=[
            pl.BlockSpec((1, _N_BLK, 1), lambda b, i: (b, i, 0)),
            pl.BlockSpec((1, 1, M), lambda b, i: (b, 0, 0)),
        ],
        out_shape=[
            jax.ShapeDtypeStruct((B, N, 1), jnp.float32),
            jax.ShapeDtypeStruct((B, 1, M), jnp.float32),
        ],
    )(input1, yt)
    return out1.reshape(B, N), out2.reshape(B, M)


# speedup vs baseline: 1.2875x; 1.2875x over previous
import jax
import jax.numpy as jnp
from jax.experimental import pallas as pl


_N_BLK = 2048


def _nnd_body(x_ref, yt_ref, d1_ref, d2_ref):
    nb = pl.program_id(1)
    x = x_ref[0]
    yt = yt_ref[0]

    n_blk = x.shape[0]
    m = yt.shape[1]
    bf16, f32 = jnp.bfloat16, jnp.float32

    xm = ((-2.0) * x).astype(bf16)
    yb = yt.astype(bf16)
    x2 = jnp.sum(x * x, axis=1, keepdims=True)
    y2 = jnp.sum(yt * yt, axis=0, keepdims=True)
    x2h = x2.astype(bf16)
    x2l = (x2 - x2h.astype(f32)).astype(bf16)
    y2h = y2.astype(bf16)
    y2l = (y2 - y2h.astype(f32)).astype(bf16)

    lhs = jnp.concatenate(
        [xm, x2h, x2l,
         jnp.ones((n_blk, 1), bf16), jnp.ones((n_blk, 1), bf16)], axis=1)
    rhs = jnp.concatenate(
        [yb, jnp.ones((1, m), bf16), jnp.ones((1, m), bf16),
         y2h, y2l], axis=0)

    d = jax.lax.dot_general(lhs, rhs, (((1,), (0,)), ((), ())),
                            preferred_element_type=f32)

    d1_ref[0] = jnp.maximum(jnp.min(d, axis=1, keepdims=True), 0.0)

    cur = jnp.maximum(jnp.min(d, axis=0, keepdims=True), 0.0)

    @pl.when(nb == 0)
    def _init():
        d2_ref[0] = cur

    @pl.when(nb != 0)
    def _accum():
        d2_ref[0] = jnp.minimum(d2_ref[0], cur)


def kernel(input1, input2):
    B, N, _ = input1.shape
    M = input2.shape[1]
    yt = jnp.transpose(input2, (0, 2, 1))

    nb = N // _N_BLK
    out1, out2 = pl.pallas_call(
        _nnd_body,
        grid=(B, nb),
        in_specs=[
            pl.BlockSpec((1, _N_BLK, 3), lambda b, i: (b, i, 0)),
            pl.BlockSpec((1, 3, M), lambda b, i: (b, 0, 0)),
        ],
        out_specs=[
            pl.BlockSpec((1, _N_BLK, 1), lambda b, i: (b, i, 0)),
            pl.BlockSpec((1, 1, M), lambda b, i: (b, 0, 0)),
        ],
        out_shape=[
            jax.ShapeDtypeStruct((B, N, 1), jnp.float32),
            jax.ShapeDtypeStruct((B, 1, M), jnp.float32),
        ],
    )(input1, yt)
    return out1.reshape(B, N), out2.reshape(B, M)
